# Initial kernel scaffold; baseline (speedup 1.0000x reference)
import jax
import jax.numpy as jnp
from jax.experimental import pallas as pl


def kernel(accumulator):
    raise NotImplementedError("write your pallas kernel here")


# speedup vs baseline: 19.9652x; 19.9652x over previous
import functools

import numpy as np
import jax
import jax.numpy as jnp
from jax.experimental import pallas as pl
from jax.experimental.pallas import tpu as pltpu

NUMANGLE = 180
NUMRHO = 184
OUT_H = 128
OUT_W = 128
P = OUT_H * OUT_W

P_TILE = 2048
A_BLK = 4


def _rho_index_table(H, W, numangle, numrho):
    irho = float(int(np.sqrt(H * H + W * W) + 1)) / float(numrho - 1)
    angles = np.arange(numangle).astype(np.float64) * (np.pi / numangle)
    cosi = np.cos(angles) / irho
    sini = np.sin(angles) / irho
    xs = (np.arange(W) - W // 2).astype(np.float64)
    ys = (np.arange(H) - H // 2).astype(np.float64)
    r = np.round(
        cosi[:, None, None] * xs[None, None, :] + sini[:, None, None] * ys[None, :, None]
    ).astype(np.int32) + numrho // 2
    invalid = (r < 0) | (r >= numrho)
    r[invalid] = numrho + 7
    return r.reshape(numangle, 1, H * W)


def _idht_block(ridx_ref, acc_ref, out_ref):
    a = pl.program_id(1)
    contrib = None
    for j in range(A_BLK):
        idx = ridx_ref[j, 0, :]
        iota = jax.lax.broadcasted_iota(jnp.int32, (NUMRHO, P_TILE), 0)
        onehot = (iota == idx[None, :]).astype(jnp.bfloat16)
        d = jnp.dot(acc_ref[j], onehot, preferred_element_type=jnp.float32)
        contrib = d if contrib is None else contrib + d

    @pl.when(a == 0)
    def _init():
        out_ref[...] = contrib

    @pl.when(a > 0)
    def _accum():
        out_ref[...] += contrib


@functools.partial(jax.jit, static_argnames=("interpret",))
def kernel(accumulator, interpret=False):
    n, c, a_dim, r_dim = accumulator.shape
    nc = n * c
    ridx = jnp.asarray(_rho_index_table(OUT_H, OUT_W, NUMANGLE, NUMRHO))
    acc_t = jnp.transpose(accumulator.reshape(nc, a_dim, r_dim), (1, 0, 2)).astype(
        jnp.bfloat16
    )

    out = pl.pallas_call(
        _idht_block,
        grid=(P // P_TILE, NUMANGLE // A_BLK),
        in_specs=[
            pl.BlockSpec((A_BLK, 1, P_TILE), lambda p, a: (a, 0, p)),
            pl.BlockSpec((A_BLK, nc, r_dim), lambda p, a: (a, 0, 0)),
        ],
        out_specs=pl.BlockSpec((nc, P_TILE), lambda p, a: (0, p)),
        out_shape=jax.ShapeDtypeStruct((nc, P), jnp.float32),
        compiler_params=pltpu.CompilerParams(
            dimension_semantics=("parallel", "arbitrary"),
        ),
        interpret=interpret,
    )(ridx, acc_t)

    return out.reshape(n, c, OUT_H, OUT_W)


# speedup vs baseline: 22.1306x; 1.1085x over previous
import functools

import numpy as np
import jax
import jax.numpy as jnp
from jax.experimental import pallas as pl
from jax.experimental.pallas import tpu as pltpu

NUMANGLE = 180
NUMRHO = 184
R_PAD = 192
OUT_H = 128
OUT_W = 128
P = OUT_H * OUT_W

P_TILE = 2048
A_BLK = 4
K = A_BLK * R_PAD


def _rho_index_table(H, W, numangle, numrho):
    irho = float(int(np.sqrt(H * H + W * W) + 1)) / float(numrho - 1)
    angles = np.arange(numangle).astype(np.float64) * (np.pi / numangle)
    cosi = np.cos(angles) / irho
    sini = np.sin(angles) / irho
    xs = (np.arange(W) - W // 2).astype(np.float64)
    ys = (np.arange(H) - H // 2).astype(np.float64)
    r = np.round(
        cosi[:, None, None] * xs[None, None, :] + sini[:, None, None] * ys[None, :, None]
    ).astype(np.int32) + numrho // 2
    invalid = (r < 0) | (r >= numrho)
    r[invalid] = numrho
    r = r.reshape(numangle, H * W)
    r += (np.arange(numangle)[:, None] % A_BLK) * R_PAD
    return r.reshape(numangle // A_BLK, A_BLK, H * W)


def _idht_block(ridx_ref, acc_ref, out_ref):
    a = pl.program_id(1)
    iota = jax.lax.broadcasted_iota(jnp.int32, (K, P_TILE), 0)
    match = iota == ridx_ref[0, 0, :][None, :]
    for j in range(1, A_BLK):
        match = match | (iota == ridx_ref[0, j, :][None, :])
    onehot = match.astype(jnp.bfloat16)
    d = jnp.dot(acc_ref[0], onehot, preferred_element_type=jnp.float32)

    @pl.when(a == 0)
    def _init():
        out_ref[...] = d

    @pl.when(a > 0)
    def _accum():
        out_ref[...] += d


@functools.partial(jax.jit, static_argnames=("interpret",))
def kernel(accumulator, interpret=False):
    n, c, a_dim, r_dim = accumulator.shape
    nc = n * c
    a_grid = a_dim // A_BLK
    ridx = jnp.asarray(_rho_index_table(OUT_H, OUT_W, NUMANGLE, NUMRHO))
    acc_p = jnp.pad(
        accumulator.reshape(nc, a_dim, r_dim), ((0, 0), (0, 0), (0, R_PAD - r_dim))
    )
    acc_g = (
        acc_p.reshape(nc, a_grid, A_BLK * R_PAD)
        .transpose(1, 0, 2)
        .astype(jnp.bfloat16)
    )

    out = pl.pallas_call(
        _idht_block,
        grid=(P // P_TILE, a_grid),
        in_specs=[
            pl.BlockSpec((1, A_BLK, P_TILE), lambda p, a: (a, 0, p)),
            pl.BlockSpec((1, nc, K), lambda p, a: (a, 0, 0)),
        ],
        out_specs=pl.BlockSpec((nc, P_TILE), lambda p, a: (0, p)),
        out_shape=jax.ShapeDtypeStruct((nc, P), jnp.float32),
        compiler_params=pltpu.CompilerParams(
            dimension_semantics=("parallel", "arbitrary"),
        ),
        interpret=interpret,
    )(ridx, acc_g)

    return out.reshape(n, c, OUT_H, OUT_W)


# speedup vs baseline: 24.0860x; 1.0884x over previous
import functools

import numpy as np
import jax
import jax.numpy as jnp
from jax.experimental import pallas as pl
from jax.experimental.pallas import tpu as pltpu

NUMANGLE = 180
NUMRHO = 184
R_PAD = 192
OUT_H = 128
OUT_W = 128
P = OUT_H * OUT_W

P_TILE = 2048
A_BLK = 4
K = A_BLK * R_PAD


def _rho_index_table(H, W, numangle, numrho):
    irho = float(int(np.sqrt(H * H + W * W) + 1)) / float(numrho - 1)
    angles = np.arange(numangle).astype(np.float64) * (np.pi / numangle)
    cosi = np.cos(angles) / irho
    sini = np.sin(angles) / irho
    xs = (np.arange(W) - W // 2).astype(np.float64)
    ys = (np.arange(H) - H // 2).astype(np.float64)
    r = np.round(
        cosi[:, None, None] * xs[None, None, :] + sini[:, None, None] * ys[None, :, None]
    ).astype(np.int32) + numrho // 2
    invalid = (r < 0) | (r >= numrho)
    r[invalid] = numrho
    return r.reshape(numangle // A_BLK, A_BLK, H * W)


def _idht_block(ridx_ref, acc_ref, out_ref, oh_ref):
    a = pl.program_id(1)
    iota = jax.lax.broadcasted_iota(jnp.int32, (R_PAD, P_TILE), 0)
    for j in range(A_BLK):
        oh_ref[j * R_PAD : (j + 1) * R_PAD, :] = (
            iota == ridx_ref[0, j, :][None, :]
        ).astype(jnp.bfloat16)
    d = jnp.dot(acc_ref[0], oh_ref[...], preferred_element_type=jnp.float32)

    @pl.when(a == 0)
    def _init():
        out_ref[...] = d

    @pl.when(a > 0)
    def _accum():
        out_ref[...] += d


@functools.partial(jax.jit, static_argnames=("interpret",))
def kernel(accumulator, interpret=False):
    n, c, a_dim, r_dim = accumulator.shape
    nc = n * c
    a_grid = a_dim // A_BLK
    ridx = jnp.asarray(_rho_index_table(OUT_H, OUT_W, NUMANGLE, NUMRHO))
    acc_p = jnp.pad(
        accumulator.reshape(nc, a_dim, r_dim), ((0, 0), (0, 0), (0, R_PAD - r_dim))
    )
    acc_g = (
        acc_p.reshape(nc, a_grid, A_BLK * R_PAD)
        .transpose(1, 0, 2)
        .astype(jnp.bfloat16)
    )

    out = pl.pallas_call(
        _idht_block,
        grid=(P // P_TILE, a_grid),
        in_specs=[
            pl.BlockSpec((1, A_BLK, P_TILE), lambda p, a: (a, 0, p)),
            pl.BlockSpec((1, nc, K), lambda p, a: (a, 0, 0)),
        ],
        out_specs=pl.BlockSpec((nc, P_TILE), lambda p, a: (0, p)),
        out_shape=jax.ShapeDtypeStruct((nc, P), jnp.float32),
        scratch_shapes=[pltpu.VMEM((K, P_TILE), jnp.bfloat16)],
        compiler_params=pltpu.CompilerParams(
            dimension_semantics=("parallel", "arbitrary"),
        ),
        interpret=interpret,
    )(ridx, acc_g)

    return out.reshape(n, c, OUT_H, OUT_W)


# speedup vs baseline: 28.2742x; 1.1739x over previous
import functools

import numpy as np
import jax
import jax.numpy as jnp
from jax.experimental import pallas as pl
from jax.experimental.pallas import tpu as pltpu

NUMANGLE = 180
NUMRHO = 184
R_PAD = 192
OUT_H = 128
OUT_W = 128
P = OUT_H * OUT_W

P_TILE = 2048
A_BLK = 12
K = A_BLK * R_PAD


def _rho_index_table(H, W, numangle, numrho):
    irho = float(int(np.sqrt(H * H + W * W) + 1)) / float(numrho - 1)
    angles = np.arange(numangle).astype(np.float64) * (np.pi / numangle)
    cosi = np.cos(angles) / irho
    sini = np.sin(angles) / irho
    xs = (np.arange(W) - W // 2).astype(np.float64)
    ys = (np.arange(H) - H // 2).astype(np.float64)
    r = np.round(
        cosi[:, None, None] * xs[None, None, :] + sini[:, None, None] * ys[None, :, None]
    ).astype(np.int32) + numrho // 2
    invalid = (r < 0) | (r >= numrho)
    r[invalid] = numrho
    return r.reshape(numangle // A_BLK, A_BLK, H * W)


def _idht_block(ridx_ref, acc_ref, out_ref, oh_ref):
    a = pl.program_id(1)
    iota = jax.lax.broadcasted_iota(jnp.int32, (R_PAD, P_TILE), 0)
    for j in range(A_BLK):
        oh_ref[j * R_PAD : (j + 1) * R_PAD, :] = (
            iota == ridx_ref[0, j, :][None, :]
        ).astype(jnp.bfloat16)
    d = jnp.dot(acc_ref[0], oh_ref[...], preferred_element_type=jnp.float32)

    @pl.when(a == 0)
    def _init():
        out_ref[...] = d

    @pl.when(a > 0)
    def _accum():
        out_ref[...] += d


@functools.partial(jax.jit, static_argnames=("interpret",))
def kernel(accumulator, interpret=False):
    n, c, a_dim, r_dim = accumulator.shape
    nc = n * c
    a_grid = a_dim // A_BLK
    ridx = jnp.asarray(_rho_index_table(OUT_H, OUT_W, NUMANGLE, NUMRHO))
    acc_p = jnp.pad(
        accumulator.reshape(nc, a_dim, r_dim), ((0, 0), (0, 0), (0, R_PAD - r_dim))
    )
    acc_g = (
        acc_p.reshape(nc, a_grid, A_BLK * R_PAD)
        .transpose(1, 0, 2)
        .astype(jnp.bfloat16)
    )

    out = pl.pallas_call(
        _idht_block,
        grid=(P // P_TILE, a_grid),
        in_specs=[
            pl.BlockSpec((1, A_BLK, P_TILE), lambda p, a: (a, 0, p)),
            pl.BlockSpec((1, nc, K), lambda p, a: (a, 0, 0)),
        ],
        out_specs=pl.BlockSpec((nc, P_TILE), lambda p, a: (0, p)),
        out_shape=jax.ShapeDtypeStruct((nc, P), jnp.float32),
        scratch_shapes=[pltpu.VMEM((K, P_TILE), jnp.bfloat16)],
        compiler_params=pltpu.CompilerParams(
            dimension_semantics=("parallel", "arbitrary"),
        ),
        interpret=interpret,
    )(ridx, acc_g)

    return out.reshape(n, c, OUT_H, OUT_W)
